# Initial kernel scaffold; baseline (speedup 1.0000x reference)
import jax
import jax.numpy as jnp
from jax.experimental import pallas as pl


def kernel(dataUser, dataItem, user_table, item_table, W1, b1, W2, b2):
    raise NotImplementedError("write your pallas kernel here")


# speedup vs baseline: 1.4048x; 1.4048x over previous
import functools

import jax
import jax.numpy as jnp
from jax import lax
from jax.experimental import pallas as pl
from jax.experimental.pallas import tpu as pltpu
from jax.experimental.pallas import tpu_sc as plsc

VOCAB = 100000
EMB = 16
BATCH = 16384
NC = 2
NS = 16
NW = NC * NS
BPW = BATCH // NW
LANES = 16
NCHUNK = BPW // LANES
IDXCH = 128
NIDX = BPW // IDXCH

OFF_W1 = 0
OFF_B1 = 1024
OFF_W2 = 1056
OFF_B2 = 1088
WPACK = 1104


def _sc_body(du_hbm, di_hbm, ut_hbm, it_hbm, wp_hbm, out_hbm,
             idx_u, idx_i, xu, xi, wv, logits_v, sem):
    c = lax.axis_index("c")
    s = lax.axis_index("s")
    wid = s * NC + c

    pltpu.sync_copy(wp_hbm, wv)
    pltpu.sync_copy(du_hbm.at[wid], idx_u)
    pltpu.sync_copy(di_hbm.at[wid], idx_i)

    cps = []
    for j in range(NIDX):
        cps.append(pltpu.async_copy(
            ut_hbm.at[idx_u.at[j]], xu.at[pl.ds(j * IDXCH, IDXCH)], sem))
        cps.append(pltpu.async_copy(
            it_hbm.at[idx_i.at[j]], xi.at[pl.ds(j * IDXCH, IDXCH)], sem))
    for cp in cps:
        cp.wait()

    riota = lax.iota(jnp.int32, LANES)
    cols = [jnp.full((LANES,), k, jnp.int32) for k in range(EMB)]

    def chunk(ci, carry):
        rows = ci * LANES + riota
        feats = [plsc.load_gather(xu, [rows, cols[k]]) for k in range(EMB)]
        feats += [plsc.load_gather(xi, [rows, cols[k]]) for k in range(EMB)]
        b1a = wv[pl.ds(OFF_B1, LANES)]
        b1b = wv[pl.ds(OFF_B1 + LANES, LANES)]
        w2a = wv[pl.ds(OFF_W2, LANES)]
        w2b = wv[pl.ds(OFF_W2 + LANES, LANES)]
        b2v = wv[pl.ds(OFF_B2, LANES)]
        acc2 = jnp.full((LANES,), 0.0, jnp.float32)
        for jf in range(32):
            wa = wv[pl.ds(OFF_W1 + jf * 32, LANES)]
            wb = wv[pl.ds(OFF_W1 + jf * 32 + LANES, LANES)]
            bj = b1a[jf] if jf < LANES else b1b[jf - LANES]
            h = jnp.full((LANES,), 0.0, jnp.float32) + bj
            for k in range(LANES):
                h = h + feats[k] * wa[k]
            for k in range(LANES):
                h = h + feats[LANES + k] * wb[k]
            h = jnp.maximum(h, 0.0)
            w2j = w2a[jf] if jf < LANES else w2b[jf - LANES]
            acc2 = acc2 + h * w2j
        acc2 = acc2 + b2v[0]
        logits_v[pl.ds(ci * LANES, LANES)] = acc2
        return carry

    lax.fori_loop(0, NCHUNK, chunk, 0)
    pltpu.sync_copy(logits_v, out_hbm.at[wid])


@jax.jit
def _run(du, di, ut, it, wpack):
    mesh = plsc.VectorSubcoreMesh(
        core_axis_name="c", subcore_axis_name="s",
        num_cores=NC, num_subcores=NS)
    f = pl.kernel(
        _sc_body,
        out_type=jax.ShapeDtypeStruct((NW, BPW), jnp.float32),
        mesh=mesh,
        compiler_params=pltpu.CompilerParams(
            needs_layout_passes=False, use_tc_tiling_on_sc=False),
        scratch_types=[
            pltpu.VMEM((NIDX, IDXCH), jnp.int32),
            pltpu.VMEM((NIDX, IDXCH), jnp.int32),
            pltpu.VMEM((BPW, EMB), jnp.float32),
            pltpu.VMEM((BPW, EMB), jnp.float32),
            pltpu.VMEM((WPACK,), jnp.float32),
            pltpu.VMEM((BPW,), jnp.float32),
            pltpu.SemaphoreType.DMA,
        ],
    )
    return f(du, di, ut, it, wpack)


def kernel(dataUser, dataItem, user_table, item_table, W1, b1, W2, b2):
    du = dataUser.astype(jnp.int32).reshape(NW, NIDX, IDXCH)
    di = dataItem.astype(jnp.int32).reshape(NW, NIDX, IDXCH)
    wpack = jnp.concatenate([
        W1.reshape(-1), b1.reshape(-1), W2.reshape(-1), b2.reshape(-1),
        jnp.zeros((WPACK - (OFF_B2 + 1),), jnp.float32)])
    out = _run(du, di, user_table, item_table, wpack)
    return out.reshape(BATCH, 1)
